# baseline (speedup 1.0000x reference)
import functools

import jax
import jax.numpy as jnp
from jax import lax
from jax.experimental import pallas as pl
from jax.experimental.pallas import tpu as pltpu
from jax.experimental.pallas import tpu_sc as plsc

N = 10000
E = 320000
F_IN = 128
H = 256

NC = 2
NS = 16
NW = NC * NS

NPAD = 10240
RPT = NPAD // NS
CH = 128
EW = 10240
EPAD = NW * EW
NCHUNK = EW // CH

_mesh = plsc.VectorSubcoreMesh(core_axis_name="c", subcore_axis_name="s")


def _edge_pipeline(src_hbm, dst_hbm, table_hbm, acc, base,
                   sall, didx, gbuf, isem, gsem, ssem):
    pltpu.sync_copy(src_hbm.at[pl.ds(base, EW)], sall)

    def idx_start(i, q):
        pltpu.async_copy(dst_hbm.at[pl.ds(base + i * CH, CH)], didx[q],
                         isem[q])

    def idx_wait(q):
        pltpu.make_async_copy(dst_hbm.at[pl.ds(base, CH)], didx[q],
                              isem[q]).wait()

    def gather_start(i, b):
        pltpu.async_copy(table_hbm.at[sall.at[pl.ds(i * CH, CH)]], gbuf[b],
                         gsem[b])

    def gather_wait(b):
        pltpu.make_async_copy(table_hbm.at[sall.at[pl.ds(0, CH)]], gbuf[b],
                              gsem[b]).wait()

    def scat_start(b, q):
        pltpu.async_copy(gbuf[b], acc.at[didx[q]], ssem[b], add=True)

    def scat_wait(b, q):
        pltpu.make_async_copy(gbuf[b], acc.at[didx[q]], ssem[b]).wait()

    def step(i, u, do_swait, do_istart):
        b = u % 2
        q = u % 4
        q2 = (u + 2) % 4
        if do_swait:
            scat_wait(b, q2)
        if do_istart:
            idx_start(i + 2, q2)
        idx_wait(q)
        gather_start(i, b)
        gather_wait(b)
        scat_start(b, q)

    for q in range(4):
        idx_start(q, q)
    step(0, 0, False, False)
    step(1, 1, False, False)
    step(2, 2, True, True)
    step(3, 3, True, True)

    def group(i4, carry):
        for u in range(4):
            step(i4 * 4 + u, u, True, True)
        return carry

    lax.fori_loop(1, NCHUNK // 4 - 1, group, 0)

    last = NCHUNK - 4
    step(last + 0, 0, True, True)
    step(last + 1, 1, True, True)
    step(last + 2, 2, True, False)
    step(last + 3, 3, True, False)
    scat_wait(0, 2)
    scat_wait(1, 3)


@functools.partial(
    pl.kernel,
    out_type=jax.ShapeDtypeStruct((NC, NPAD, F_IN), jnp.float32),
    mesh=_mesh,
    scratch_types=[
        pltpu.VMEM((EW,), jnp.int32),
        [pltpu.VMEM((CH,), jnp.int32)] * 4,
        [pltpu.VMEM((CH, F_IN), jnp.float32)] * 2,
        [pltpu.SemaphoreType.DMA] * 4,
        [pltpu.SemaphoreType.DMA] * 2,
        [pltpu.SemaphoreType.DMA] * 2,
        pltpu.VMEM_SHARED((NPAD, F_IN), jnp.float32),
    ],
)
def _agg_kernel(x_hbm, src_hbm, dst_hbm, zero_hbm, out_hbm,
                sall, didx, gbuf, isem, gsem, ssem, acc):
    c = lax.axis_index("c")
    s = lax.axis_index("s")
    wid = s * NC + c

    pltpu.sync_copy(zero_hbm, acc.at[pl.ds(s * RPT, RPT)])
    plsc.subcore_barrier()

    _edge_pipeline(src_hbm, dst_hbm, x_hbm, acc, wid * EW,
                   sall, didx, gbuf, isem, gsem, ssem)
    plsc.subcore_barrier()

    pltpu.sync_copy(acc.at[pl.ds(s * RPT, RPT)], out_hbm.at[c, pl.ds(s * RPT, RPT)])


@functools.partial(
    pl.kernel,
    out_type=jax.ShapeDtypeStruct((NC, NPAD), jnp.float32),
    mesh=_mesh,
    scratch_types=[
        pltpu.VMEM((EW,), jnp.int32),
        [pltpu.VMEM((CH,), jnp.int32)] * 4,
        [pltpu.VMEM((CH,), jnp.float32)] * 2,
        [pltpu.SemaphoreType.DMA] * 4,
        [pltpu.SemaphoreType.DMA] * 2,
        [pltpu.SemaphoreType.DMA] * 2,
        pltpu.VMEM_SHARED((NPAD,), jnp.float32),
    ],
)
def _score_kernel(pr_hbm, src_hbm, dst_hbm, zero1_hbm, out_hbm,
                  sall, didx, vals, isem, gsem, ssem, acc):
    c = lax.axis_index("c")
    s = lax.axis_index("s")
    wid = s * NC + c

    pltpu.sync_copy(zero1_hbm, acc.at[pl.ds(s * RPT, RPT)])
    plsc.subcore_barrier()

    _edge_pipeline(src_hbm, dst_hbm, pr_hbm, acc, wid * EW,
                   sall, didx, vals, isem, gsem, ssem)
    plsc.subcore_barrier()

    pltpu.sync_copy(acc.at[pl.ds(s * RPT, RPT)], out_hbm.at[c, pl.ds(s * RPT, RPT)])


_BLK = 2048


def _mm_body(x_ref, p0_ref, p1_ref, wr_ref, wo_ref, b_ref, wpr_ref, wpo_ref,
             bp1_ref, h_ref, pr_ref, po_ref):
    agg = p0_ref[...] + p1_ref[...]
    h = jnp.dot(agg, wr_ref[...], preferred_element_type=jnp.float32,
                precision=lax.Precision.HIGHEST)
    h += jnp.dot(x_ref[...], wo_ref[...], preferred_element_type=jnp.float32,
                 precision=lax.Precision.HIGHEST)
    h = jnp.maximum(h + b_ref[...], 0.0)
    h_ref[...] = h
    pr_ref[...] = jnp.sum(h * wpr_ref[...], axis=1)
    po_ref[...] = jnp.sum(h * wpo_ref[...], axis=1) + bp1_ref[0, 0]


_mm_call = pl.pallas_call(
    _mm_body,
    grid=(NPAD // _BLK,),
    in_specs=[
        pl.BlockSpec((_BLK, F_IN), lambda i: (i, 0)),
        pl.BlockSpec((_BLK, F_IN), lambda i: (i, 0)),
        pl.BlockSpec((_BLK, F_IN), lambda i: (i, 0)),
        pl.BlockSpec((F_IN, H), lambda i: (0, 0)),
        pl.BlockSpec((F_IN, H), lambda i: (0, 0)),
        pl.BlockSpec((1, H), lambda i: (0, 0)),
        pl.BlockSpec((1, H), lambda i: (0, 0)),
        pl.BlockSpec((1, H), lambda i: (0, 0)),
        pl.BlockSpec((1, 1), lambda i: (0, 0)),
    ],
    out_specs=[
        pl.BlockSpec((_BLK, H), lambda i: (i, 0)),
        pl.BlockSpec((_BLK,), lambda i: (i,)),
        pl.BlockSpec((_BLK,), lambda i: (i,)),
    ],
    out_shape=[
        jax.ShapeDtypeStruct((NPAD, H), jnp.float32),
        jax.ShapeDtypeStruct((NPAD,), jnp.float32),
        jax.ShapeDtypeStruct((NPAD,), jnp.float32),
    ],
)


def _fin_body(sc_ref, po_ref, ei_ref, h_ref,
              wr2_ref, wo2_ref, b2_ref, wpr2_ref, wpo2_ref, bp2_ref,
              wr3_ref, wo3_ref, b3_ref, wpr3_ref, wpo3_ref, bp3_ref,
              wm_ref, bm_ref, out_ref, yrow, sem):
    s = sc_ref[0:1, :] + sc_ref[1:2, :] + po_ref[...]
    col = lax.broadcasted_iota(jnp.int32, (1, NPAD), 1)
    s = jnp.where(col < N, s, -jnp.inf)
    v = jnp.max(s)
    p = jnp.min(jnp.where(s >= v, col, NPAD))

    cp = pltpu.make_async_copy(h_ref.at[pl.ds(p, 1)], yrow, sem)
    cp.start()
    cp.wait()
    y1 = yrow[...] * jnp.tanh(jnp.full((1, 1), v, jnp.float32))

    cnt = jnp.sum(jnp.where((ei_ref[0] == p) & (ei_ref[1] == p), 1.0, 0.0))

    def layer(y, wr, wo, b, wpr, wpo, bp):
        z = cnt * jnp.dot(y, wr, preferred_element_type=jnp.float32)
        z += jnp.dot(y, wo, preferred_element_type=jnp.float32)
        z = jnp.maximum(z + b, 0.0)
        s2 = cnt * jnp.sum(z * wpr) + jnp.sum(z * wpo) + bp
        return z * jnp.tanh(jnp.full((1, 1), s2, jnp.float32))

    y2 = layer(y1, wr2_ref[...], wo2_ref[...], b2_ref[...],
               wpr2_ref[...], wpo2_ref[...], bp2_ref[0, 0])
    y3 = layer(y2, wr3_ref[...], wo3_ref[...], b3_ref[...],
               wpr3_ref[...], wpo3_ref[...], bp3_ref[0, 0])
    ys = y1 + y2 + y3
    out_ref[...] = jnp.dot(ys, wm_ref[...],
                           preferred_element_type=jnp.float32) + bm_ref[...]


_fin_call = pl.pallas_call(
    _fin_body,
    in_specs=[
        pl.BlockSpec(memory_space=pltpu.VMEM),
        pl.BlockSpec(memory_space=pltpu.VMEM),
        pl.BlockSpec(memory_space=pltpu.VMEM),
        pl.BlockSpec(memory_space=pl.ANY),
    ] + [pl.BlockSpec(memory_space=pltpu.VMEM)] * 14,
    out_specs=pl.BlockSpec(memory_space=pltpu.VMEM),
    out_shape=jax.ShapeDtypeStruct((1, 2), jnp.float32),
    scratch_shapes=[
        pltpu.VMEM((1, H), jnp.float32),
        pltpu.SemaphoreType.DMA,
    ],
)


def kernel(x, edge_index, batch, Wr1, Wo1, b1, Wpr1, Wpo1, bp1,
           Wr2, Wo2, b2, Wpr2, Wpo2, bp2, Wr3, Wo3, b3, Wpr3, Wpo3, bp3,
           Wm, bm):
    src = edge_index[0]
    dst = edge_index[1]

    srcp = jnp.concatenate([src, jnp.zeros((EPAD - E,), jnp.int32)])
    dstp = jnp.concatenate([dst, jnp.full((EPAD - E,), N, jnp.int32)])

    xp = jnp.pad(x, ((0, NPAD - N), (0, 0)))
    parts = _agg_kernel(xp, srcp, dstp, jnp.zeros((RPT, F_IN), jnp.float32))

    h, pr, po = _mm_call(
        xp, parts[0], parts[1], Wr1, Wo1, b1.reshape(1, H),
        Wpr1.reshape(1, H), Wpo1.reshape(1, H), bp1.reshape(1, 1))

    sc = _score_kernel(pr, srcp, dstp, jnp.zeros((RPT,), jnp.float32))

    ei3 = edge_index.reshape(2, E // 128, 128)
    out = _fin_call(
        sc, po.reshape(1, NPAD), ei3, h,
        Wr2, Wo2, b2.reshape(1, H), Wpr2.reshape(1, H), Wpo2.reshape(1, H),
        bp2.reshape(1, 1),
        Wr3, Wo3, b3.reshape(1, H), Wpr3.reshape(1, H), Wpo3.reshape(1, H),
        bp3.reshape(1, 1),
        Wm[:H] + Wm[H:], bm.reshape(1, 2))
    return out


# speedup vs baseline: 1.0370x; 1.0370x over previous
import functools

import jax
import jax.numpy as jnp
from jax import lax
from jax.experimental import pallas as pl
from jax.experimental.pallas import tpu as pltpu
from jax.experimental.pallas import tpu_sc as plsc

N = 10000
E = 320000
F_IN = 128
H = 256

NC = 2
NS = 16
NW = NC * NS

NPAD = 10240
RPT = NPAD // NS
CH = 128
EW = 10240
EPAD = NW * EW
NCHUNK = EW // CH

_mesh = plsc.VectorSubcoreMesh(core_axis_name="c", subcore_axis_name="s")


def _edge_pipeline(src_hbm, dst_hbm, table_hbm, acc, base,
                   sall, didx, gbuf, isem, gsem, ssem):
    pltpu.sync_copy(src_hbm.at[pl.ds(base, EW)], sall)

    def idx_start(i, q):
        pltpu.async_copy(dst_hbm.at[pl.ds(base + i * CH, CH)], didx[q],
                         isem[q])

    def idx_wait(q):
        pltpu.make_async_copy(dst_hbm.at[pl.ds(base, CH)], didx[q],
                              isem[q]).wait()

    def gather_start(i, b):
        pltpu.async_copy(table_hbm.at[sall.at[pl.ds(i * CH, CH)]], gbuf[b],
                         gsem[b])

    def gather_wait(b):
        pltpu.make_async_copy(table_hbm.at[sall.at[pl.ds(0, CH)]], gbuf[b],
                              gsem[b]).wait()

    def scat_start(b, q):
        pltpu.async_copy(gbuf[b], acc.at[didx[q]], ssem[b], add=True)

    def scat_wait(b, q):
        pltpu.make_async_copy(gbuf[b], acc.at[didx[q]], ssem[b]).wait()

    def step(i, u, do_swait, do_istart):
        b = u % 2
        q = u % 4
        q2 = (u + 2) % 4
        if do_swait:
            scat_wait(b, q2)
        if do_istart:
            idx_start(i + 2, q2)
        idx_wait(q)
        gather_start(i, b)
        gather_wait(b)
        scat_start(b, q)

    for q in range(4):
        idx_start(q, q)
    step(0, 0, False, False)
    step(1, 1, False, False)
    step(2, 2, True, True)
    step(3, 3, True, True)

    def group(i4, carry):
        for u in range(4):
            step(i4 * 4 + u, u, True, True)
        return carry

    lax.fori_loop(1, NCHUNK // 4 - 1, group, 0)

    last = NCHUNK - 4
    step(last + 0, 0, True, True)
    step(last + 1, 1, True, True)
    step(last + 2, 2, True, False)
    step(last + 3, 3, True, False)
    scat_wait(0, 2)
    scat_wait(1, 3)


@functools.partial(
    pl.kernel,
    out_type=jax.ShapeDtypeStruct((NC, NPAD, F_IN), jnp.float32),
    mesh=_mesh,
    scratch_types=[
        pltpu.VMEM((EW,), jnp.int32),
        [pltpu.VMEM((CH,), jnp.int32)] * 4,
        [pltpu.VMEM((CH, F_IN), jnp.float32)] * 2,
        [pltpu.SemaphoreType.DMA] * 4,
        [pltpu.SemaphoreType.DMA] * 2,
        [pltpu.SemaphoreType.DMA] * 2,
        pltpu.VMEM_SHARED((NPAD, F_IN), jnp.float32),
    ],
)
def _agg_kernel(x_hbm, src_hbm, dst_hbm, zero_hbm, out_hbm,
                sall, didx, gbuf, isem, gsem, ssem, acc):
    c = lax.axis_index("c")
    s = lax.axis_index("s")
    wid = s * NC + c

    pltpu.sync_copy(zero_hbm, acc.at[pl.ds(s * RPT, RPT)])
    plsc.subcore_barrier()

    _edge_pipeline(src_hbm, dst_hbm, x_hbm, acc, wid * EW,
                   sall, didx, gbuf, isem, gsem, ssem)
    plsc.subcore_barrier()

    pltpu.sync_copy(acc.at[pl.ds(s * RPT, RPT)], out_hbm.at[c, pl.ds(s * RPT, RPT)])


@functools.partial(
    pl.kernel,
    out_type=jax.ShapeDtypeStruct((NC, NPAD), jnp.float32),
    mesh=_mesh,
    scratch_types=[
        pltpu.VMEM((EW,), jnp.int32),
        [pltpu.VMEM((CH,), jnp.int32)] * 4,
        [pltpu.VMEM((CH,), jnp.float32)] * 2,
        [pltpu.SemaphoreType.DMA] * 4,
        [pltpu.SemaphoreType.DMA] * 2,
        [pltpu.SemaphoreType.DMA] * 2,
        pltpu.VMEM_SHARED((NPAD,), jnp.float32),
    ],
)
def _score_kernel(pr_hbm, src_hbm, dst_hbm, zero1_hbm, out_hbm,
                  sall, didx, vals, isem, gsem, ssem, acc):
    c = lax.axis_index("c")
    s = lax.axis_index("s")
    wid = s * NC + c

    pltpu.sync_copy(zero1_hbm, acc.at[pl.ds(s * RPT, RPT)])
    plsc.subcore_barrier()

    _edge_pipeline(src_hbm, dst_hbm, pr_hbm, acc, wid * EW,
                   sall, didx, vals, isem, gsem, ssem)
    plsc.subcore_barrier()

    pltpu.sync_copy(acc.at[pl.ds(s * RPT, RPT)], out_hbm.at[c, pl.ds(s * RPT, RPT)])


_BLK = 2048


def _mm_body(x_ref, p0_ref, p1_ref, wr_ref, wo_ref, b_ref, wpr_ref, wpo_ref,
             bp1_ref, h_ref, pr_ref, po_ref):
    agg = p0_ref[...] + p1_ref[...]
    h = jnp.dot(agg.astype(jnp.bfloat16), wr_ref[...].astype(jnp.bfloat16),
                preferred_element_type=jnp.float32)
    h += jnp.dot(x_ref[...].astype(jnp.bfloat16),
                 wo_ref[...].astype(jnp.bfloat16),
                 preferred_element_type=jnp.float32)
    h = jnp.maximum(h + b_ref[...], 0.0)
    h_ref[...] = h
    pr_ref[...] = jnp.sum(h * wpr_ref[...], axis=1)
    po_ref[...] = jnp.sum(h * wpo_ref[...], axis=1) + bp1_ref[0, 0]


_mm_call = pl.pallas_call(
    _mm_body,
    grid=(NPAD // _BLK,),
    in_specs=[
        pl.BlockSpec((_BLK, F_IN), lambda i: (i, 0)),
        pl.BlockSpec((_BLK, F_IN), lambda i: (i, 0)),
        pl.BlockSpec((_BLK, F_IN), lambda i: (i, 0)),
        pl.BlockSpec((F_IN, H), lambda i: (0, 0)),
        pl.BlockSpec((F_IN, H), lambda i: (0, 0)),
        pl.BlockSpec((1, H), lambda i: (0, 0)),
        pl.BlockSpec((1, H), lambda i: (0, 0)),
        pl.BlockSpec((1, H), lambda i: (0, 0)),
        pl.BlockSpec((1, 1), lambda i: (0, 0)),
    ],
    out_specs=[
        pl.BlockSpec((_BLK, H), lambda i: (i, 0)),
        pl.BlockSpec((_BLK,), lambda i: (i,)),
        pl.BlockSpec((_BLK,), lambda i: (i,)),
    ],
    out_shape=[
        jax.ShapeDtypeStruct((NPAD, H), jnp.float32),
        jax.ShapeDtypeStruct((NPAD,), jnp.float32),
        jax.ShapeDtypeStruct((NPAD,), jnp.float32),
    ],
)


def _fin_body(sc_ref, po_ref, ei_ref, h_ref,
              wr2_ref, wo2_ref, b2_ref, wpr2_ref, wpo2_ref, bp2_ref,
              wr3_ref, wo3_ref, b3_ref, wpr3_ref, wpo3_ref, bp3_ref,
              wm_ref, bm_ref, out_ref, yrow, sem):
    s = sc_ref[0:1, :] + sc_ref[1:2, :] + po_ref[...]
    col = lax.broadcasted_iota(jnp.int32, (1, NPAD), 1)
    s = jnp.where(col < N, s, -jnp.inf)
    v = jnp.max(s)
    p = jnp.min(jnp.where(s >= v, col, NPAD))

    cp = pltpu.make_async_copy(h_ref.at[pl.ds(p, 1)], yrow, sem)
    cp.start()
    cp.wait()
    y1 = yrow[...] * jnp.tanh(jnp.full((1, 1), v, jnp.float32))

    cnt = jnp.sum(jnp.where((ei_ref[0] == p) & (ei_ref[1] == p), 1.0, 0.0))

    def layer(y, wr, wo, b, wpr, wpo, bp):
        z = cnt * jnp.dot(y, wr, preferred_element_type=jnp.float32)
        z += jnp.dot(y, wo, preferred_element_type=jnp.float32)
        z = jnp.maximum(z + b, 0.0)
        s2 = cnt * jnp.sum(z * wpr) + jnp.sum(z * wpo) + bp
        return z * jnp.tanh(jnp.full((1, 1), s2, jnp.float32))

    y2 = layer(y1, wr2_ref[...], wo2_ref[...], b2_ref[...],
               wpr2_ref[...], wpo2_ref[...], bp2_ref[0, 0])
    y3 = layer(y2, wr3_ref[...], wo3_ref[...], b3_ref[...],
               wpr3_ref[...], wpo3_ref[...], bp3_ref[0, 0])
    ys = y1 + y2 + y3
    out_ref[...] = jnp.dot(ys, wm_ref[...],
                           preferred_element_type=jnp.float32) + bm_ref[...]


_fin_call = pl.pallas_call(
    _fin_body,
    in_specs=[
        pl.BlockSpec(memory_space=pltpu.VMEM),
        pl.BlockSpec(memory_space=pltpu.VMEM),
        pl.BlockSpec(memory_space=pltpu.VMEM),
        pl.BlockSpec(memory_space=pl.ANY),
    ] + [pl.BlockSpec(memory_space=pltpu.VMEM)] * 14,
    out_specs=pl.BlockSpec(memory_space=pltpu.VMEM),
    out_shape=jax.ShapeDtypeStruct((1, 2), jnp.float32),
    scratch_shapes=[
        pltpu.VMEM((1, H), jnp.float32),
        pltpu.SemaphoreType.DMA,
    ],
)


def kernel(x, edge_index, batch, Wr1, Wo1, b1, Wpr1, Wpo1, bp1,
           Wr2, Wo2, b2, Wpr2, Wpo2, bp2, Wr3, Wo3, b3, Wpr3, Wpo3, bp3,
           Wm, bm):
    src = edge_index[0]
    dst = edge_index[1]

    srcp = jnp.concatenate([src, jnp.zeros((EPAD - E,), jnp.int32)])
    dstp = jnp.concatenate([dst, jnp.full((EPAD - E,), N, jnp.int32)])

    xp = jnp.pad(x, ((0, NPAD - N), (0, 0)))
    parts = _agg_kernel(xp, srcp, dstp, jnp.zeros((RPT, F_IN), jnp.float32))

    h, pr, po = _mm_call(
        xp, parts[0], parts[1], Wr1, Wo1, b1.reshape(1, H),
        Wpr1.reshape(1, H), Wpo1.reshape(1, H), bp1.reshape(1, 1))

    sc = _score_kernel(pr, srcp, dstp, jnp.zeros((RPT,), jnp.float32))

    ei3 = edge_index.reshape(2, E // 128, 128)
    out = _fin_call(
        sc, po.reshape(1, NPAD), ei3, h,
        Wr2, Wo2, b2.reshape(1, H), Wpr2.reshape(1, H), Wpo2.reshape(1, H),
        bp2.reshape(1, 1),
        Wr3, Wo3, b3.reshape(1, H), Wpr3.reshape(1, H), Wpo3.reshape(1, H),
        bp3.reshape(1, 1),
        Wm[:H] + Wm[H:], bm.reshape(1, 2))
    return out


# speedup vs baseline: 1.1841x; 1.1418x over previous
import functools

import jax
import jax.numpy as jnp
from jax import lax
from jax.experimental import pallas as pl
from jax.experimental.pallas import tpu as pltpu
from jax.experimental.pallas import tpu_sc as plsc

N = 10000
E = 320000
F_IN = 128
H = 256

NC = 2
NS = 16
NW = NC * NS

NPAD = 10240
RPT = NPAD // NS
CH = 128
NCH0 = 120
NCH1 = 40
EW0 = NCH0 * CH
EW1 = NCH1 * CH
OFF1 = NS * EW0
EPAD = NS * (EW0 + EW1)
SALL = max(EW0, EW1)
EALLOC = max((NS - 1) * EW0, OFF1 + (NS - 1) * EW1) + SALL
EALLOC = max(EALLOC, EPAD)

_mesh = plsc.VectorSubcoreMesh(core_axis_name="c", subcore_axis_name="s")


def _edge_pipeline(src_hbm, dst_hbm, table_hbm, acc, base, nch,
                   sall, didx, gbuf, isem, gsem, ssem):
    pltpu.sync_copy(src_hbm.at[pl.ds(base, SALL)], sall)

    def idx_start(i, q):
        pltpu.async_copy(dst_hbm.at[pl.ds(base + i * CH, CH)], didx[q],
                         isem[q])

    def idx_wait(q):
        pltpu.make_async_copy(dst_hbm.at[pl.ds(base, CH)], didx[q],
                              isem[q]).wait()

    def gather_start(i, b):
        pltpu.async_copy(table_hbm.at[sall.at[pl.ds(i * CH, CH)]], gbuf[b],
                         gsem[b])

    def gather_wait(b):
        pltpu.make_async_copy(table_hbm.at[sall.at[pl.ds(0, CH)]], gbuf[b],
                              gsem[b]).wait()

    def scat_start(b, q):
        pltpu.async_copy(gbuf[b], acc.at[didx[q]], ssem[b], add=True)

    def scat_wait(b, q):
        pltpu.make_async_copy(gbuf[b], acc.at[didx[q]], ssem[b]).wait()

    def step(i, u, do_swait, do_istart):
        b = u % 2
        q = u % 4
        q2 = (u + 2) % 4
        if do_swait:
            scat_wait(b, q2)
        if do_istart:
            idx_start(i + 2, q2)
        idx_wait(q)
        gather_start(i, b)
        gather_wait(b)
        scat_start(b, q)

    for q in range(4):
        idx_start(q, q)
    step(0, 0, False, False)
    step(1, 1, False, False)
    step(2, 2, True, True)
    step(3, 3, True, True)

    def group(i4, carry):
        for u in range(4):
            step(i4 * 4 + u, u, True, True)
        return carry

    lax.fori_loop(1, nch // 4 - 1, group, 0)

    last = nch - 4
    step(last + 0, 0, True, True)
    step(last + 1, 1, True, True)
    step(last + 2, 2, True, False)
    step(last + 3, 3, True, False)
    scat_wait(0, 2)
    scat_wait(1, 3)


@functools.partial(
    pl.kernel,
    out_type=jax.ShapeDtypeStruct((NC, NPAD, F_IN), jnp.float32),
    mesh=_mesh,
    scratch_types=[
        pltpu.VMEM((SALL,), jnp.int32),
        [pltpu.VMEM((CH,), jnp.int32)] * 4,
        [pltpu.VMEM((CH, F_IN), jnp.float32)] * 2,
        [pltpu.SemaphoreType.DMA] * 4,
        [pltpu.SemaphoreType.DMA] * 2,
        [pltpu.SemaphoreType.DMA] * 2,
        pltpu.VMEM_SHARED((NPAD, F_IN), jnp.float32),
    ],
)
def _agg_kernel(x_hbm, src_hbm, dst_hbm, zero_hbm, out_hbm,
                sall, didx, gbuf, isem, gsem, ssem, acc):
    c = lax.axis_index("c")
    s = lax.axis_index("s")
    base = jnp.where(c == 0, s * EW0, OFF1 + s * EW1)
    nch = jnp.where(c == 0, NCH0, NCH1)

    pltpu.sync_copy(zero_hbm, acc.at[pl.ds(s * RPT, RPT)])
    plsc.subcore_barrier()

    _edge_pipeline(src_hbm, dst_hbm, x_hbm, acc, base, nch,
                   sall, didx, gbuf, isem, gsem, ssem)
    plsc.subcore_barrier()

    pltpu.sync_copy(acc.at[pl.ds(s * RPT, RPT)], out_hbm.at[c, pl.ds(s * RPT, RPT)])


@functools.partial(
    pl.kernel,
    out_type=jax.ShapeDtypeStruct((NC, NPAD), jnp.float32),
    mesh=_mesh,
    scratch_types=[
        pltpu.VMEM((SALL,), jnp.int32),
        [pltpu.VMEM((CH,), jnp.int32)] * 4,
        [pltpu.VMEM((CH,), jnp.float32)] * 2,
        [pltpu.SemaphoreType.DMA] * 4,
        [pltpu.SemaphoreType.DMA] * 2,
        [pltpu.SemaphoreType.DMA] * 2,
        pltpu.VMEM_SHARED((NPAD,), jnp.float32),
    ],
)
def _score_kernel(pr_hbm, src_hbm, dst_hbm, zero1_hbm, out_hbm,
                  sall, didx, vals, isem, gsem, ssem, acc):
    c = lax.axis_index("c")
    s = lax.axis_index("s")
    base = jnp.where(c == 0, s * EW0, OFF1 + s * EW1)
    nch = jnp.where(c == 0, NCH0, NCH1)

    pltpu.sync_copy(zero1_hbm, acc.at[pl.ds(s * RPT, RPT)])
    plsc.subcore_barrier()

    _edge_pipeline(src_hbm, dst_hbm, pr_hbm, acc, base, nch,
                   sall, didx, vals, isem, gsem, ssem)
    plsc.subcore_barrier()

    pltpu.sync_copy(acc.at[pl.ds(s * RPT, RPT)], out_hbm.at[c, pl.ds(s * RPT, RPT)])


_BLK = 2048


def _mm_body(x_ref, p0_ref, p1_ref, wr_ref, wo_ref, b_ref, wpr_ref, wpo_ref,
             bp1_ref, h_ref, pr_ref, po_ref):
    agg = p0_ref[...] + p1_ref[...]
    h = jnp.dot(agg.astype(jnp.bfloat16), wr_ref[...].astype(jnp.bfloat16),
                preferred_element_type=jnp.float32)
    h += jnp.dot(x_ref[...].astype(jnp.bfloat16),
                 wo_ref[...].astype(jnp.bfloat16),
                 preferred_element_type=jnp.float32)
    h = jnp.maximum(h + b_ref[...], 0.0)
    h_ref[...] = h
    pr_ref[...] = jnp.sum(h * wpr_ref[...], axis=1)
    po_ref[...] = jnp.sum(h * wpo_ref[...], axis=1) + bp1_ref[0, 0]


_mm_call = pl.pallas_call(
    _mm_body,
    grid=(NPAD // _BLK,),
    in_specs=[
        pl.BlockSpec((_BLK, F_IN), lambda i: (i, 0)),
        pl.BlockSpec((_BLK, F_IN), lambda i: (i, 0)),
        pl.BlockSpec((_BLK, F_IN), lambda i: (i, 0)),
        pl.BlockSpec((F_IN, H), lambda i: (0, 0)),
        pl.BlockSpec((F_IN, H), lambda i: (0, 0)),
        pl.BlockSpec((1, H), lambda i: (0, 0)),
        pl.BlockSpec((1, H), lambda i: (0, 0)),
        pl.BlockSpec((1, H), lambda i: (0, 0)),
        pl.BlockSpec((1, 1), lambda i: (0, 0)),
    ],
    out_specs=[
        pl.BlockSpec((_BLK, H), lambda i: (i, 0)),
        pl.BlockSpec((_BLK,), lambda i: (i,)),
        pl.BlockSpec((_BLK,), lambda i: (i,)),
    ],
    out_shape=[
        jax.ShapeDtypeStruct((NPAD, H), jnp.float32),
        jax.ShapeDtypeStruct((NPAD,), jnp.float32),
        jax.ShapeDtypeStruct((NPAD,), jnp.float32),
    ],
)


def _fin_body(sc_ref, po_ref, ei_ref, h_ref,
              wr2_ref, wo2_ref, b2_ref, wpr2_ref, wpo2_ref, bp2_ref,
              wr3_ref, wo3_ref, b3_ref, wpr3_ref, wpo3_ref, bp3_ref,
              wm_ref, bm_ref, out_ref, yrow, sem):
    s = sc_ref[0:1, :] + sc_ref[1:2, :] + po_ref[...]
    col = lax.broadcasted_iota(jnp.int32, (1, NPAD), 1)
    s = jnp.where(col < N, s, -jnp.inf)
    v = jnp.max(s)
    p = jnp.min(jnp.where(s >= v, col, NPAD))

    cp = pltpu.make_async_copy(h_ref.at[pl.ds(p, 1)], yrow, sem)
    cp.start()
    cp.wait()
    y1 = yrow[...] * jnp.tanh(jnp.full((1, 1), v, jnp.float32))

    cnt = jnp.sum(jnp.where((ei_ref[0] == p) & (ei_ref[1] == p), 1.0, 0.0))

    def layer(y, wr, wo, b, wpr, wpo, bp):
        z = cnt * jnp.dot(y, wr, preferred_element_type=jnp.float32)
        z += jnp.dot(y, wo, preferred_element_type=jnp.float32)
        z = jnp.maximum(z + b, 0.0)
        s2 = cnt * jnp.sum(z * wpr) + jnp.sum(z * wpo) + bp
        return z * jnp.tanh(jnp.full((1, 1), s2, jnp.float32))

    y2 = layer(y1, wr2_ref[...], wo2_ref[...], b2_ref[...],
               wpr2_ref[...], wpo2_ref[...], bp2_ref[0, 0])
    y3 = layer(y2, wr3_ref[...], wo3_ref[...], b3_ref[...],
               wpr3_ref[...], wpo3_ref[...], bp3_ref[0, 0])
    ys = y1 + y2 + y3
    out_ref[...] = jnp.dot(ys, wm_ref[...],
                           preferred_element_type=jnp.float32) + bm_ref[...]


_fin_call = pl.pallas_call(
    _fin_body,
    in_specs=[
        pl.BlockSpec(memory_space=pltpu.VMEM),
        pl.BlockSpec(memory_space=pltpu.VMEM),
        pl.BlockSpec(memory_space=pltpu.VMEM),
        pl.BlockSpec(memory_space=pl.ANY),
    ] + [pl.BlockSpec(memory_space=pltpu.VMEM)] * 14,
    out_specs=pl.BlockSpec(memory_space=pltpu.VMEM),
    out_shape=jax.ShapeDtypeStruct((1, 2), jnp.float32),
    scratch_shapes=[
        pltpu.VMEM((1, H), jnp.float32),
        pltpu.SemaphoreType.DMA,
    ],
)


def kernel(x, edge_index, batch, Wr1, Wo1, b1, Wpr1, Wpo1, bp1,
           Wr2, Wo2, b2, Wpr2, Wpo2, bp2, Wr3, Wo3, b3, Wpr3, Wpo3, bp3,
           Wm, bm):
    src = edge_index[0]
    dst = edge_index[1]

    srcp = jnp.concatenate([src, jnp.zeros((EALLOC - E,), jnp.int32)])
    dstp = jnp.concatenate([dst, jnp.full((EALLOC - E,), N, jnp.int32)])

    xp = jnp.pad(x, ((0, NPAD - N), (0, 0)))
    parts = _agg_kernel(xp, srcp, dstp, jnp.zeros((RPT, F_IN), jnp.float32))

    h, pr, po = _mm_call(
        xp, parts[0], parts[1], Wr1, Wo1, b1.reshape(1, H),
        Wpr1.reshape(1, H), Wpo1.reshape(1, H), bp1.reshape(1, 1))

    sc = _score_kernel(pr, srcp, dstp, jnp.zeros((RPT,), jnp.float32))

    ei3 = edge_index.reshape(2, E // 128, 128)
    out = _fin_call(
        sc, po.reshape(1, NPAD), ei3, h,
        Wr2, Wo2, b2.reshape(1, H), Wpr2.reshape(1, H), Wpo2.reshape(1, H),
        bp2.reshape(1, 1),
        Wr3, Wo3, b3.reshape(1, H), Wpr3.reshape(1, H), Wpo3.reshape(1, H),
        bp3.reshape(1, 1),
        Wm[:H] + Wm[H:], bm.reshape(1, 2))
    return out


# speedup vs baseline: 1.2436x; 1.0503x over previous
import functools

import jax
import jax.numpy as jnp
from jax import lax
from jax.experimental import pallas as pl
from jax.experimental.pallas import tpu as pltpu
from jax.experimental.pallas import tpu_sc as plsc

N = 10000
E = 320000
F_IN = 128
H = 256

NC = 2
NS = 16
NW = NC * NS

NPAD = 10240
RPT = NPAD // NS
CH = 128
NCH0 = 124
NCH1 = 36
EW0 = NCH0 * CH
EW1 = NCH1 * CH
OFF1 = NS * EW0
EPAD = NS * (EW0 + EW1)
SALL = max(EW0, EW1)
EALLOC = max((NS - 1) * EW0, OFF1 + (NS - 1) * EW1) + SALL
EALLOC = max(EALLOC, EPAD)

_mesh = plsc.VectorSubcoreMesh(core_axis_name="c", subcore_axis_name="s")


def _edge_pipeline(src_hbm, dst_hbm, table_hbm, acc, base, nch,
                   sall, didx, gbuf, isem, gsem, ssem):
    pltpu.sync_copy(src_hbm.at[pl.ds(base, SALL)], sall)

    def idx_start(i, q):
        pltpu.async_copy(dst_hbm.at[pl.ds(base + i * CH, CH)], didx[q],
                         isem[q])

    def idx_wait(q):
        pltpu.make_async_copy(dst_hbm.at[pl.ds(base, CH)], didx[q],
                              isem[q]).wait()

    def gather_start(i, b):
        pltpu.async_copy(table_hbm.at[sall.at[pl.ds(i * CH, CH)]], gbuf[b],
                         gsem[b])

    def gather_wait(b):
        pltpu.make_async_copy(table_hbm.at[sall.at[pl.ds(0, CH)]], gbuf[b],
                              gsem[b]).wait()

    def scat_start(b, q):
        pltpu.async_copy(gbuf[b], acc.at[didx[q]], ssem[b], add=True)

    def scat_wait(b, q):
        pltpu.make_async_copy(gbuf[b], acc.at[didx[q]], ssem[b]).wait()

    def step(i, u, do_swait, do_istart):
        b = u % 2
        q = u % 4
        q2 = (u + 2) % 4
        if do_swait:
            scat_wait(b, q2)
        if do_istart:
            idx_start(i + 2, q2)
        idx_wait(q)
        gather_start(i, b)
        gather_wait(b)
        scat_start(b, q)

    for q in range(4):
        idx_start(q, q)
    step(0, 0, False, False)
    step(1, 1, False, False)
    step(2, 2, True, True)
    step(3, 3, True, True)

    def group(i4, carry):
        for u in range(4):
            step(i4 * 4 + u, u, True, True)
        return carry

    lax.fori_loop(1, nch // 4 - 1, group, 0)

    last = nch - 4
    step(last + 0, 0, True, True)
    step(last + 1, 1, True, True)
    step(last + 2, 2, True, False)
    step(last + 3, 3, True, False)
    scat_wait(0, 2)
    scat_wait(1, 3)


@functools.partial(
    pl.kernel,
    out_type=jax.ShapeDtypeStruct((NC, NPAD, F_IN), jnp.float32),
    mesh=_mesh,
    scratch_types=[
        pltpu.VMEM((SALL,), jnp.int32),
        [pltpu.VMEM((CH,), jnp.int32)] * 4,
        [pltpu.VMEM((CH, F_IN), jnp.float32)] * 2,
        [pltpu.SemaphoreType.DMA] * 4,
        [pltpu.SemaphoreType.DMA] * 2,
        [pltpu.SemaphoreType.DMA] * 2,
        pltpu.VMEM_SHARED((NPAD, F_IN), jnp.float32),
    ],
)
def _agg_kernel(x_hbm, src_hbm, dst_hbm, zero_hbm, out_hbm,
                sall, didx, gbuf, isem, gsem, ssem, acc):
    c = lax.axis_index("c")
    s = lax.axis_index("s")
    base = jnp.where(c == 0, s * EW0, OFF1 + s * EW1)
    nch = jnp.where(c == 0, NCH0, NCH1)

    pltpu.sync_copy(zero_hbm, acc.at[pl.ds(s * RPT, RPT)])
    plsc.subcore_barrier()

    _edge_pipeline(src_hbm, dst_hbm, x_hbm, acc, base, nch,
                   sall, didx, gbuf, isem, gsem, ssem)
    plsc.subcore_barrier()

    pltpu.sync_copy(acc.at[pl.ds(s * RPT, RPT)], out_hbm.at[c, pl.ds(s * RPT, RPT)])


@functools.partial(
    pl.kernel,
    out_type=jax.ShapeDtypeStruct((NC, NPAD), jnp.float32),
    mesh=_mesh,
    scratch_types=[
        pltpu.VMEM((SALL,), jnp.int32),
        [pltpu.VMEM((CH,), jnp.int32)] * 4,
        [pltpu.VMEM((CH,), jnp.float32)] * 2,
        [pltpu.SemaphoreType.DMA] * 4,
        [pltpu.SemaphoreType.DMA] * 2,
        [pltpu.SemaphoreType.DMA] * 2,
        pltpu.VMEM_SHARED((NPAD,), jnp.float32),
    ],
)
def _score_kernel(pr_hbm, src_hbm, dst_hbm, zero1_hbm, out_hbm,
                  sall, didx, vals, isem, gsem, ssem, acc):
    c = lax.axis_index("c")
    s = lax.axis_index("s")
    wid = s * NC + c
    ew = EPAD // NW
    base = wid * ew
    nch = ew // CH

    pltpu.sync_copy(zero1_hbm, acc.at[pl.ds(s * RPT, RPT)])
    plsc.subcore_barrier()

    _edge_pipeline(src_hbm, dst_hbm, pr_hbm, acc, base, nch,
                   sall, didx, vals, isem, gsem, ssem)
    plsc.subcore_barrier()

    pltpu.sync_copy(acc.at[pl.ds(s * RPT, RPT)], out_hbm.at[c, pl.ds(s * RPT, RPT)])


_BLK = 2048


def _mm_body(x_ref, p0_ref, p1_ref, wr_ref, wo_ref, b_ref, wpr_ref, wpo_ref,
             bp1_ref, h_ref, pr_ref, po_ref):
    agg = p0_ref[...] + p1_ref[...]
    h = jnp.dot(agg.astype(jnp.bfloat16), wr_ref[...].astype(jnp.bfloat16),
                preferred_element_type=jnp.float32)
    h += jnp.dot(x_ref[...].astype(jnp.bfloat16),
                 wo_ref[...].astype(jnp.bfloat16),
                 preferred_element_type=jnp.float32)
    h = jnp.maximum(h + b_ref[...], 0.0)
    h_ref[...] = h
    pr_ref[...] = jnp.sum(h * wpr_ref[...], axis=1)
    po_ref[...] = jnp.sum(h * wpo_ref[...], axis=1) + bp1_ref[0, 0]


_mm_call = pl.pallas_call(
    _mm_body,
    grid=(NPAD // _BLK,),
    in_specs=[
        pl.BlockSpec((_BLK, F_IN), lambda i: (i, 0)),
        pl.BlockSpec((_BLK, F_IN), lambda i: (i, 0)),
        pl.BlockSpec((_BLK, F_IN), lambda i: (i, 0)),
        pl.BlockSpec((F_IN, H), lambda i: (0, 0)),
        pl.BlockSpec((F_IN, H), lambda i: (0, 0)),
        pl.BlockSpec((1, H), lambda i: (0, 0)),
        pl.BlockSpec((1, H), lambda i: (0, 0)),
        pl.BlockSpec((1, H), lambda i: (0, 0)),
        pl.BlockSpec((1, 1), lambda i: (0, 0)),
    ],
    out_specs=[
        pl.BlockSpec((_BLK, H), lambda i: (i, 0)),
        pl.BlockSpec((_BLK,), lambda i: (i,)),
        pl.BlockSpec((_BLK,), lambda i: (i,)),
    ],
    out_shape=[
        jax.ShapeDtypeStruct((NPAD, H), jnp.float32),
        jax.ShapeDtypeStruct((NPAD,), jnp.float32),
        jax.ShapeDtypeStruct((NPAD,), jnp.float32),
    ],
)


def _fin_body(sc_ref, po_ref, ei_ref, h_ref,
              wr2_ref, wo2_ref, b2_ref, wpr2_ref, wpo2_ref, bp2_ref,
              wr3_ref, wo3_ref, b3_ref, wpr3_ref, wpo3_ref, bp3_ref,
              wm_ref, bm_ref, out_ref, yrow, sem):
    s = sc_ref[0:1, :] + sc_ref[1:2, :] + po_ref[...]
    col = lax.broadcasted_iota(jnp.int32, (1, NPAD), 1)
    s = jnp.where(col < N, s, -jnp.inf)
    v = jnp.max(s)
    p = jnp.min(jnp.where(s >= v, col, NPAD))

    cp = pltpu.make_async_copy(h_ref.at[pl.ds(p, 1)], yrow, sem)
    cp.start()
    cp.wait()
    y1 = yrow[...] * jnp.tanh(jnp.full((1, 1), v, jnp.float32))

    cnt = jnp.sum(jnp.where((ei_ref[0] == p) & (ei_ref[1] == p), 1.0, 0.0))

    def layer(y, wr, wo, b, wpr, wpo, bp):
        z = cnt * jnp.dot(y, wr, preferred_element_type=jnp.float32)
        z += jnp.dot(y, wo, preferred_element_type=jnp.float32)
        z = jnp.maximum(z + b, 0.0)
        s2 = cnt * jnp.sum(z * wpr) + jnp.sum(z * wpo) + bp
        return z * jnp.tanh(jnp.full((1, 1), s2, jnp.float32))

    y2 = layer(y1, wr2_ref[...], wo2_ref[...], b2_ref[...],
               wpr2_ref[...], wpo2_ref[...], bp2_ref[0, 0])
    y3 = layer(y2, wr3_ref[...], wo3_ref[...], b3_ref[...],
               wpr3_ref[...], wpo3_ref[...], bp3_ref[0, 0])
    ys = y1 + y2 + y3
    out_ref[...] = jnp.dot(ys, wm_ref[...],
                           preferred_element_type=jnp.float32) + bm_ref[...]


_fin_call = pl.pallas_call(
    _fin_body,
    in_specs=[
        pl.BlockSpec(memory_space=pltpu.VMEM),
        pl.BlockSpec(memory_space=pltpu.VMEM),
        pl.BlockSpec(memory_space=pltpu.VMEM),
        pl.BlockSpec(memory_space=pl.ANY),
    ] + [pl.BlockSpec(memory_space=pltpu.VMEM)] * 14,
    out_specs=pl.BlockSpec(memory_space=pltpu.VMEM),
    out_shape=jax.ShapeDtypeStruct((1, 2), jnp.float32),
    scratch_shapes=[
        pltpu.VMEM((1, H), jnp.float32),
        pltpu.SemaphoreType.DMA,
    ],
)


def kernel(x, edge_index, batch, Wr1, Wo1, b1, Wpr1, Wpo1, bp1,
           Wr2, Wo2, b2, Wpr2, Wpo2, bp2, Wr3, Wo3, b3, Wpr3, Wpo3, bp3,
           Wm, bm):
    src = edge_index[0]
    dst = edge_index[1]

    srcp = jnp.concatenate([src, jnp.zeros((EALLOC - E,), jnp.int32)])
    dstp = jnp.concatenate([dst, jnp.full((EALLOC - E,), N, jnp.int32)])

    xp = jnp.pad(x, ((0, NPAD - N), (0, 0)))
    parts = _agg_kernel(xp, srcp, dstp, jnp.zeros((RPT, F_IN), jnp.float32))

    h, pr, po = _mm_call(
        xp, parts[0], parts[1], Wr1, Wo1, b1.reshape(1, H),
        Wpr1.reshape(1, H), Wpo1.reshape(1, H), bp1.reshape(1, 1))

    sc = _score_kernel(pr, srcp, dstp, jnp.zeros((RPT,), jnp.float32))

    ei3 = edge_index.reshape(2, E // 128, 128)
    out = _fin_call(
        sc, po.reshape(1, NPAD), ei3, h,
        Wr2, Wo2, b2.reshape(1, H), Wpr2.reshape(1, H), Wpo2.reshape(1, H),
        bp2.reshape(1, 1),
        Wr3, Wo3, b3.reshape(1, H), Wpr3.reshape(1, H), Wpo3.reshape(1, H),
        bp3.reshape(1, 1),
        Wm[:H] + Wm[H:], bm.reshape(1, 2))
    return out
